# baseline (speedup 1.0000x reference)
import functools

import jax
import jax.numpy as jnp
import numpy as np
from jax.experimental import pallas as pl

_ROTATIONS = ([13, 15, 26, 6], [17, 29, 16, 24])


def _threefry_constants():
    def rotl(x, d):
        x = np.uint32(x)
        return np.uint32((np.uint64(x) << np.uint64(d)) & np.uint64(0xFFFFFFFF)) | np.uint32(
            x >> np.uint32(32 - d))

    def add(a, b):
        return np.uint32((np.uint64(a) + np.uint64(b)) & np.uint64(0xFFFFFFFF))

    k1, k2 = np.uint32(0), np.uint32(42)
    ks = [k1, k2, np.uint32(k1 ^ k2 ^ np.uint32(0x1BD11BDA))]
    x = [add(0, ks[0]), add(7, ks[1])]
    for i in range(5):
        for r in _ROTATIONS[i % 2]:
            x[0] = add(x[0], x[1])
            x[1] = rotl(x[1], r)
            x[1] = np.uint32(x[0] ^ x[1])
        x[0] = add(x[0], ks[(i + 1) % 3])
        x[1] = add(add(x[1], ks[(i + 2) % 3]), i + 1)
    return x[0], x[1]


_K1, _K2 = _threefry_constants()


def _sample_block(prior_ref, out_ref, *, by: int, ny: int, nx: int, nc: int):
    yb = pl.program_id(0)
    b = pl.program_id(1)

    logits = prior_ref[0]

    k1 = np.uint32(_K1)
    k2 = np.uint32(_K2)
    k3 = np.uint32(k1 ^ k2 ^ np.uint32(0x1BD11BDA))
    ks = (k1, k2, k3)

    shp = (by, nx)
    y_i = jax.lax.broadcasted_iota(jnp.uint32, shp, 0)
    x_i = jax.lax.broadcasted_iota(jnp.uint32, shp, 1)
    base = (b.astype(jnp.uint32) * np.uint32(ny) + jnp.uint32(by) * yb.astype(jnp.uint32)) \
        * np.uint32(nx * nc) + k2
    vox = (y_i * np.uint32(nx) + x_i) * np.uint32(nc)
    tiny = np.float32(np.finfo(np.float32).tiny)

    best = None
    idx_best = None
    for c in range(nc):
        x0 = None
        x1 = vox + (base + np.uint32(c))
        for i in range(5):
            for r in _ROTATIONS[i % 2]:
                x0 = x0 + x1 if x0 is not None else x1 + k1
                x1 = (x1 << np.uint32(r)) | (x1 >> np.uint32(32 - r))
                x1 = x0 ^ x1
            x0 = x0 + ks[(i + 1) % 3]
            x1 = x1 + np.uint32((int(ks[(i + 2) % 3]) + i + 1) & 0xFFFFFFFF)
        bits = x0 ^ x1

        float_bits = (bits >> np.uint32(9)) | np.uint32(0x3F800000)
        floats = jax.lax.bitcast_convert_type(float_bits, jnp.float32) - np.float32(1.0)
        u = jnp.maximum(floats, tiny)
        score = logits[c] - jnp.log(-jnp.log(u))
        if best is None:
            best = score
            idx_best = jnp.zeros(shp, dtype=jnp.int32)
        else:
            m = score > best
            idx_best = jnp.where(m, np.int32(c), idx_best)
            best = jnp.where(m, score, best)
    out_ref[0, 0] = idx_best


@jax.jit
def kernel(prior):
    nb, nc, ny, nx = prior.shape
    by = 128
    grid = (ny // by, nb)
    return pl.pallas_call(
        functools.partial(_sample_block, by=by, ny=ny, nx=nx, nc=nc),
        grid=grid,
        in_specs=[pl.BlockSpec((1, nc, by, nx), lambda y, b: (b, 0, y, 0))],
        out_specs=pl.BlockSpec((1, 1, by, nx), lambda y, b: (b, 0, y, 0)),
        out_shape=jax.ShapeDtypeStruct((nb, 1, ny, nx), jnp.int32),
    )(prior)


# speedup vs baseline: 1.0001x; 1.0001x over previous
import functools

import jax
import jax.numpy as jnp
import numpy as np
from jax.experimental import pallas as pl

_ROTATIONS = ([13, 15, 26, 6], [17, 29, 16, 24])


def _threefry_constants():
    def rotl(x, d):
        x = np.uint32(x)
        return np.uint32((np.uint64(x) << np.uint64(d)) & np.uint64(0xFFFFFFFF)) | np.uint32(
            x >> np.uint32(32 - d))

    def add(a, b):
        return np.uint32((np.uint64(a) + np.uint64(b)) & np.uint64(0xFFFFFFFF))

    k1, k2 = np.uint32(0), np.uint32(42)
    ks = [k1, k2, np.uint32(k1 ^ k2 ^ np.uint32(0x1BD11BDA))]
    x = [add(0, ks[0]), add(7, ks[1])]
    for i in range(5):
        for r in _ROTATIONS[i % 2]:
            x[0] = add(x[0], x[1])
            x[1] = rotl(x[1], r)
            x[1] = np.uint32(x[0] ^ x[1])
        x[0] = add(x[0], ks[(i + 1) % 3])
        x[1] = add(add(x[1], ks[(i + 2) % 3]), i + 1)
    return x[0], x[1]


_K1, _K2 = _threefry_constants()


def _sample_block(prior_ref, out_ref, *, by: int, ny: int, nx: int, nc: int):
    yb = pl.program_id(0)
    b = pl.program_id(1)

    logits = prior_ref[0]

    k1 = np.uint32(_K1)
    k2 = np.uint32(_K2)
    k3 = np.uint32(k1 ^ k2 ^ np.uint32(0x1BD11BDA))
    ks = (k1, k2, k3)

    shp = (by, nx)
    y_i = jax.lax.broadcasted_iota(jnp.uint32, shp, 0)
    x_i = jax.lax.broadcasted_iota(jnp.uint32, shp, 1)
    base = (b.astype(jnp.uint32) * np.uint32(ny) + jnp.uint32(by) * yb.astype(jnp.uint32)) \
        * np.uint32(nx * nc) + k2
    vox = (y_i * np.uint32(nx) + x_i) * np.uint32(nc)
    tiny = np.float32(np.finfo(np.float32).tiny)

    def chain(c):
        x0 = None
        x1 = vox + (base + np.uint32(c))
        for i in range(5):
            for r in _ROTATIONS[i % 2]:
                x0 = x0 + x1 if x0 is not None else x1 + k1
                x1 = (x1 << np.uint32(r)) | (x1 >> np.uint32(32 - r))
                x1 = x0 ^ x1
            x0 = x0 + ks[(i + 1) % 3]
            x1 = x1 + np.uint32((int(ks[(i + 2) % 3]) + i + 1) & 0xFFFFFFFF)
        bits = x0 ^ x1

        float_bits = (bits >> np.uint32(9)) | np.uint32(0x3F800000)
        floats = jax.lax.bitcast_convert_type(float_bits, jnp.float32) - np.float32(1.0)
        u = jnp.maximum(floats, tiny)
        return logits[c] - jnp.log(-jnp.log(u))

    best = None
    idx_best = None
    for c in range(0, nc, 2):
        s0 = chain(c)
        s1 = chain(c + 1)
        if best is None:
            best = s0
            idx_best = jnp.zeros(shp, dtype=jnp.int32)
        else:
            m = s0 > best
            idx_best = jnp.where(m, np.int32(c), idx_best)
            best = jnp.where(m, s0, best)
        m = s1 > best
        idx_best = jnp.where(m, np.int32(c + 1), idx_best)
        best = jnp.where(m, s1, best)
    out_ref[0, 0] = idx_best


@jax.jit
def kernel(prior):
    nb, nc, ny, nx = prior.shape
    by = 128
    grid = (ny // by, nb)
    return pl.pallas_call(
        functools.partial(_sample_block, by=by, ny=ny, nx=nx, nc=nc),
        grid=grid,
        in_specs=[pl.BlockSpec((1, nc, by, nx), lambda y, b: (b, 0, y, 0))],
        out_specs=pl.BlockSpec((1, 1, by, nx), lambda y, b: (b, 0, y, 0)),
        out_shape=jax.ShapeDtypeStruct((nb, 1, ny, nx), jnp.int32),
    )(prior)
